# baseline (speedup 1.0000x reference)
import functools

import jax
import jax.numpy as jnp
from jax import lax
from jax.experimental import pallas as pl
from jax.experimental.pallas import tpu as pltpu
from jax.experimental.pallas import tpu_sc as plsc

NC = 2
NS = 16
NW = NC * NS
LANES = 16
SUB = 128
CHUNK = 128
OUTW = 128


def _rsqrt(x):
    i = plsc.bitcast(x, jnp.int32)
    i = jnp.int32(0x5F3759DF) - (i >> 1)
    y = plsc.bitcast(i, jnp.float32)
    half_x = jnp.float32(0.5) * x
    for _ in range(3):
        y = y * (jnp.float32(1.5) - half_x * y * y)
    return y


@functools.lru_cache(maxsize=None)
def _build_sc_partials(n, m, d):
    rows_per_w = n // NW
    assert n % (NW * 2 * CHUNK) == 0 and d == 64
    n_chunks = rows_per_w // CHUNK
    n_loop = n_chunks // 2
    groups = CHUNK // LANES
    idx_rows = rows_per_w // SUB

    mesh = plsc.VectorSubcoreMesh(core_axis_name="c", subcore_axis_name="s")

    @functools.partial(
        pl.kernel,
        out_type=jax.ShapeDtypeStruct((NW * OUTW,), jnp.float32),
        mesh=mesh,
        scratch_types=[
            pltpu.VMEM((idx_rows, SUB), jnp.int32),
            pltpu.VMEM((idx_rows, SUB), jnp.int32),
            pltpu.VMEM((2, d, CHUNK), jnp.float32),
            pltpu.VMEM((2, CHUNK, 2 * d), jnp.float32),
            pltpu.VMEM((OUTW,), jnp.float32),
            pltpu.SemaphoreType.DMA,
            pltpu.SemaphoreType.DMA,
        ],
        compiler_params=pltpu.CompilerParams(
            needs_layout_passes=False, use_tc_tiling_on_sc=True),
    )
    def sc_partials(map_hbm, pred_hbm, tgt_hbm, out_hbm,
                    idx_v, idx2_v, pred_v, tgt_v, acc_v, sem0, sem1):
        wid = lax.axis_index("s") * NC + lax.axis_index("c")
        base = wid * rows_per_w
        lane_iota = lax.iota(jnp.int32, LANES)
        sems = [sem0, sem1]

        pltpu.sync_copy(map_hbm.at[pl.ds(wid * idx_rows, idx_rows)], idx_v)

        def halve_row(r, _):
            for c in range(SUB // LANES):
                idx2_v[r, pl.ds(c * LANES, LANES)] = (
                    idx_v[r, pl.ds(c * LANES, LANES)] >> 1)
            return 0

        lax.fori_loop(0, idx_rows, halve_row, 0)

        def start_chunk(c, phase):
            pltpu.async_copy(
                tgt_hbm.at[idx2_v.at[c]],
                tgt_v.at[phase],
                sems[phase],
            )
            pltpu.async_copy(
                pred_hbm.at[pl.ds(0, d), pl.ds(base + c * CHUNK, CHUNK)],
                pred_v.at[phase],
                sems[phase],
            )

        def wait_chunk(phase):
            pltpu.make_async_copy(
                tgt_hbm.at[pl.ds(0, CHUNK)],
                tgt_v.at[phase],
                sems[phase],
            ).wait()
            pltpu.make_async_copy(
                pred_hbm.at[pl.ds(0, d), pl.ds(0, CHUNK)],
                pred_v.at[phase],
                sems[phase],
            ).wait()

        def compute(phase, c, accs):
            pred_b = pred_v.at[phase]
            tgt_b = tgt_v.at[phase]

            def group_body(g, accs2):
                d_a, c_a = accs2
                rows_t = g * LANES + lane_iota
                par64t = (idx_v[c, pl.ds(g * LANES, LANES)] & 1) * d
                acc = [jnp.zeros((LANES,), jnp.float32) for _ in range(6)]
                for j in range(d):
                    diag = (lane_iota + j) & (d - 1)
                    p = plsc.load_gather(pred_b, [diag, rows_t])
                    t = plsc.load_gather(tgt_b, [rows_t, par64t + diag])
                    k = j & 1
                    acc[k] = acc[k] + p * t
                    acc[2 + k] = acc[2 + k] + p * p
                    acc[4 + k] = acc[4 + k] + t * t
                dot = acc[0] + acc[1]
                pn = acc[2] + acc[3]
                tn = acc[4] + acc[5]
                valid = jnp.logical_and(pn >= jnp.float32(1e-12),
                                        tn >= jnp.float32(1e-12))
                denom2 = jnp.where(valid, pn * tn, jnp.float32(1.0))
                dist = jnp.float32(1.0) - dot * _rsqrt(denom2)
                d_a = d_a + jnp.where(valid, dist, jnp.float32(0.0))
                c_a = c_a + jnp.where(valid, jnp.float32(1.0),
                                      jnp.float32(0.0))
                return (d_a, c_a)

            return lax.fori_loop(0, groups, group_body, accs)

        start_chunk(0, 0)

        def body(ci2, accs):
            c0 = 2 * ci2
            start_chunk(c0 + 1, 1)
            wait_chunk(0)
            accs = compute(0, c0, accs)

            @pl.when(ci2 < n_loop - 1)
            def _():
                start_chunk(c0 + 2, 0)

            wait_chunk(1)
            accs = compute(1, c0 + 1, accs)
            return accs

        zeros = jnp.zeros((LANES,), jnp.float32)
        dist_a, cnt_a = lax.fori_loop(0, n_loop, body, (zeros, zeros))
        acc_v[pl.ds(0, LANES)] = dist_a
        acc_v[pl.ds(LANES, LANES)] = cnt_a
        for c in range(2, OUTW // LANES):
            acc_v[pl.ds(c * LANES, LANES)] = zeros
        pltpu.sync_copy(acc_v, out_hbm.at[pl.ds(wid * OUTW, OUTW)])

    return sc_partials


def _finalize_body(p_ref, o_ref):
    p = p_ref[...]
    dist = jnp.sum(p[:, :LANES])
    cnt = jnp.sum(p[:, LANES:2 * LANES])
    o_ref[0, 0] = dist / jnp.maximum(cnt, jnp.float32(1.0))


_finalize = pl.pallas_call(
    _finalize_body,
    out_shape=jax.ShapeDtypeStruct((1, 1), jnp.float32),
    out_specs=pl.BlockSpec(memory_space=pltpu.SMEM),
)


def kernel(mapping, prediction, target):
    n, d = prediction.shape
    m = target.shape[0]
    mapping = mapping.astype(jnp.int32).reshape(n // SUB, SUB)
    tgt2 = target.reshape(m // 2, 2 * d)
    partials = _build_sc_partials(n, m, d)(mapping, prediction.T, tgt2)
    return _finalize(partials.reshape(NW, OUTW))[0, 0]


# speedup vs baseline: 1.2644x; 1.2644x over previous
import functools

import jax
import jax.numpy as jnp
from jax import lax
from jax.experimental import pallas as pl
from jax.experimental.pallas import tpu as pltpu
from jax.experimental.pallas import tpu_sc as plsc

NC = 2
NS = 16
NW = NC * NS
LANES = 16
SUB = 128
CHUNK = 128
OUTW = 128


def _rsqrt(x):
    i = plsc.bitcast(x, jnp.int32)
    i = jnp.int32(0x5F3759DF) - (i >> 1)
    y = plsc.bitcast(i, jnp.float32)
    half_x = jnp.float32(0.5) * x
    for _ in range(3):
        y = y * (jnp.float32(1.5) - half_x * y * y)
    return y


@functools.lru_cache(maxsize=None)
def _build_sc_partials(n, m, d):
    rows_per_w = n // NW
    assert n % (NW * 2 * CHUNK) == 0 and d == 64
    n_chunks = rows_per_w // CHUNK
    n_loop = n_chunks // 2
    groups = CHUNK // LANES
    idx_rows = rows_per_w // SUB

    mesh = plsc.VectorSubcoreMesh(core_axis_name="c", subcore_axis_name="s")

    @functools.partial(
        pl.kernel,
        out_type=jax.ShapeDtypeStruct((NW * OUTW,), jnp.float32),
        mesh=mesh,
        scratch_types=[
            pltpu.VMEM((idx_rows, SUB), jnp.int32),
            pltpu.VMEM((idx_rows, SUB), jnp.int32),
            pltpu.VMEM((2, d, CHUNK), jnp.float32),
            pltpu.VMEM((2, CHUNK, 2 * d), jnp.float32),
            pltpu.VMEM((OUTW,), jnp.float32),
            pltpu.SemaphoreType.DMA,
            pltpu.SemaphoreType.DMA,
        ],
        compiler_params=pltpu.CompilerParams(
            needs_layout_passes=False, use_tc_tiling_on_sc=True),
    )
    def sc_partials(map_hbm, pred_hbm, tgt_hbm, out_hbm,
                    idx_v, idx2_v, pred_v, tgt_v, acc_v, sem0, sem1):
        wid = lax.axis_index("s") * NC + lax.axis_index("c")
        base = wid * rows_per_w
        lane_iota = lax.iota(jnp.int32, LANES)
        sems = [sem0, sem1]

        pltpu.sync_copy(map_hbm.at[pl.ds(wid * idx_rows, idx_rows)], idx_v)

        def halve_row(r, _):
            for c in range(SUB // LANES):
                idx2_v[r, pl.ds(c * LANES, LANES)] = (
                    idx_v[r, pl.ds(c * LANES, LANES)] >> 1)
            return 0

        lax.fori_loop(0, idx_rows, halve_row, 0)

        def start_chunk(c, phase):
            pltpu.async_copy(
                tgt_hbm.at[idx2_v.at[c]],
                tgt_v.at[phase],
                sems[phase],
            )
            pltpu.async_copy(
                pred_hbm.at[pl.ds(0, d), pl.ds(base + c * CHUNK, CHUNK)],
                pred_v.at[phase],
                sems[phase],
            )

        def wait_chunk(phase):
            pltpu.make_async_copy(
                tgt_hbm.at[pl.ds(0, CHUNK)],
                tgt_v.at[phase],
                sems[phase],
            ).wait()
            pltpu.make_async_copy(
                pred_hbm.at[pl.ds(0, d), pl.ds(0, CHUNK)],
                pred_v.at[phase],
                sems[phase],
            ).wait()

        def compute(phase, c, accs):
            pred_b = pred_v.at[phase]
            tgt_b = tgt_v.at[phase]

            def group_body(g, accs2):
                d_a, c_a = accs2
                rows_t = g * LANES + lane_iota
                par64t = (idx_v[c, pl.ds(g * LANES, LANES)] & 1) * d
                dot = jnp.zeros((LANES,), jnp.float32)
                pn = jnp.zeros((LANES,), jnp.float32)
                tn = jnp.zeros((LANES,), jnp.float32)
                for j in range(d):
                    diag = (lane_iota + j) & (d - 1)
                    p = plsc.load_gather(pred_b, [diag, rows_t])
                    t = plsc.load_gather(tgt_b, [rows_t, par64t + diag])
                    dot = dot + p * t
                    pn = pn + p * p
                    tn = tn + t * t
                valid = jnp.logical_and(pn >= jnp.float32(1e-12),
                                        tn >= jnp.float32(1e-12))
                denom2 = jnp.where(valid, pn * tn, jnp.float32(1.0))
                dist = jnp.float32(1.0) - dot * _rsqrt(denom2)
                d_a = d_a + jnp.where(valid, dist, jnp.float32(0.0))
                c_a = c_a + jnp.where(valid, jnp.float32(1.0),
                                      jnp.float32(0.0))
                return (d_a, c_a)

            return lax.fori_loop(0, groups, group_body, accs)

        start_chunk(0, 0)

        def body(ci2, accs):
            c0 = 2 * ci2
            start_chunk(c0 + 1, 1)
            wait_chunk(0)
            accs = compute(0, c0, accs)

            @pl.when(ci2 < n_loop - 1)
            def _():
                start_chunk(c0 + 2, 0)

            wait_chunk(1)
            accs = compute(1, c0 + 1, accs)
            return accs

        zeros = jnp.zeros((LANES,), jnp.float32)
        dist_a, cnt_a = lax.fori_loop(0, n_loop, body, (zeros, zeros))
        acc_v[pl.ds(0, LANES)] = dist_a
        acc_v[pl.ds(LANES, LANES)] = cnt_a
        for c in range(2, OUTW // LANES):
            acc_v[pl.ds(c * LANES, LANES)] = zeros
        pltpu.sync_copy(acc_v, out_hbm.at[pl.ds(wid * OUTW, OUTW)])

    return sc_partials


def _finalize_body(p_ref, o_ref):
    p = p_ref[...]
    dist = jnp.sum(p[:, :LANES])
    cnt = jnp.sum(p[:, LANES:2 * LANES])
    o_ref[0, 0] = dist / jnp.maximum(cnt, jnp.float32(1.0))


_finalize = pl.pallas_call(
    _finalize_body,
    out_shape=jax.ShapeDtypeStruct((1, 1), jnp.float32),
    out_specs=pl.BlockSpec(memory_space=pltpu.SMEM),
)


def kernel(mapping, prediction, target):
    n, d = prediction.shape
    m = target.shape[0]
    mapping = mapping.astype(jnp.int32).reshape(n // SUB, SUB)
    tgt2 = target.reshape(m // 2, 2 * d)
    partials = _build_sc_partials(n, m, d)(mapping, prediction.T, tgt2)
    return _finalize(partials.reshape(NW, OUTW))[0, 0]
